# Initial kernel scaffold; baseline (speedup 1.0000x reference)
import jax
import jax.numpy as jnp
from jax.experimental import pallas as pl


def kernel(features, RL_thresholds, params, n_ids_0, n_ids_1, src_r0_l0, dst_r0_l0, src_r0_l1, dst_r0_l1, src_r0_l2, dst_r0_l2, src_r1_l0, dst_r1_l0, src_r1_l1, dst_r1_l1, src_r1_l2, dst_r1_l2):
    raise NotImplementedError("write your pallas kernel here")


# speedup vs baseline: 17.7374x; 17.7374x over previous
import functools
import math

import jax
import jax.numpy as jnp
from jax import lax
from jax.experimental import pallas as pl
from jax.experimental.pallas import tpu as pltpu
from jax.experimental.pallas import tpu_sc as plsc

NC, NS = 2, 16
NW = NC * NS
CH = 128

N_NODES = 100000
D = 128
N0, N1, N2, NB = 50000, 20000, 5000, 1024
E_SIZES = [320000, 80000, 16384]
EPAD = [323584, 81920, 16384]
TGT = [N1, N2, NB]
NDT = [12032, 6144, 2048]
L0_SPLIT = 10240
LAYER_SHAPES = [(128, 64, 2), (128, 64, 1), (64, 128, 1)]
IDPAD = 51200


def _mesh():
    return plsc.VectorSubcoreMesh(core_axis_name="c", subcore_axis_name="s")


def _sc_gather(table, idx):
    nch = idx.shape[0] // CH
    Dd = table.shape[1]
    npt = nch // NW

    def body(t_hbm, i_hbm, o_hbm, idxb, rows, sem):
        c = lax.axis_index("c")
        s = lax.axis_index("s")
        wid = s * NC + c

        def step(j, carry):
            row = wid * npt + j
            pltpu.sync_copy(i_hbm.at[pl.ds(row * CH, CH)], idxb)
            pltpu.async_copy(t_hbm.at[idxb], rows, sem).wait()
            pltpu.sync_copy(rows, o_hbm.at[pl.ds(row * CH, CH)])
            return carry

        lax.fori_loop(0, npt, step, 0)

    return pl.kernel(
        body,
        out_type=jax.ShapeDtypeStruct((nch * CH, Dd), jnp.float32),
        mesh=_mesh(),
        scratch_types=[
            pltpu.VMEM((CH,), jnp.int32),
            pltpu.VMEM((CH, Dd), jnp.float32),
            pltpu.SemaphoreType.DMA,
        ],
    )(table, idx)


def _sc_gather3(qt, kt, vt, dst1, src1):
    nch = dst1.shape[0] // CH
    hc = qt.shape[1]
    npt = nch // NW
    oshape = jax.ShapeDtypeStruct((nch * CH, hc), jnp.float32)

    def body(q_hbm, k_hbm, v_hbm, d_hbm, s_hbm, qd_hbm, ks_hbm, vs_hbm,
             di, si, qb, kb, vb, sem):
        c = lax.axis_index("c")
        s = lax.axis_index("s")
        wid = s * NC + c

        def step(j, carry):
            row = wid * npt + j
            pltpu.sync_copy(d_hbm.at[pl.ds(row * CH, CH)], di)
            pltpu.sync_copy(s_hbm.at[pl.ds(row * CH, CH)], si)
            cq = pltpu.async_copy(q_hbm.at[di], qb, sem)
            ck = pltpu.async_copy(k_hbm.at[si], kb, sem)
            cv = pltpu.async_copy(v_hbm.at[si], vb, sem)
            cq.wait()
            ck.wait()
            cv.wait()
            e0 = row * CH
            pltpu.sync_copy(qb, qd_hbm.at[pl.ds(e0, CH)])
            pltpu.sync_copy(kb, ks_hbm.at[pl.ds(e0, CH)])
            pltpu.sync_copy(vb, vs_hbm.at[pl.ds(e0, CH)])
            return carry

        lax.fori_loop(0, npt, step, 0)

    return pl.kernel(
        body,
        out_type=[oshape, oshape, oshape],
        mesh=_mesh(),
        scratch_types=[
            pltpu.VMEM((CH,), jnp.int32),
            pltpu.VMEM((CH,), jnp.int32),
            pltpu.VMEM((CH, hc), jnp.float32),
            pltpu.VMEM((CH, hc), jnp.float32),
            pltpu.VMEM((CH, hc), jnp.float32),
            pltpu.SemaphoreType.DMA,
        ],
    )(qt, kt, vt, dst1, src1)


def _sc_scatter(pay, dst_flat, zchunk, nd_t, w):
    eh = pay.shape[1]
    npt = eh // (NS * CH)
    rpt = nd_t // NS
    nz = rpt // CH
    rem = rpt % CH

    def body(p_hbm, d_hbm, z_hbm, o_hbm, idxb, payb, zb, shared, sem):
        c = lax.axis_index("c")
        s = lax.axis_index("s")
        pltpu.sync_copy(z_hbm, zb)

        def zstep(i, carry):
            pltpu.sync_copy(zb, shared.at[pl.ds(s * rpt + i * CH, CH)])
            return carry

        lax.fori_loop(0, nz, zstep, 0)
        if rem:
            pltpu.sync_copy(zb.at[pl.ds(0, rem)],
                            shared.at[pl.ds(s * rpt + nz * CH, rem)])
        plsc.subcore_barrier()

        def step(j, carry):
            row = s * npt + j
            pltpu.sync_copy(d_hbm.at[pl.ds(c * eh + row * CH, CH)],
                            idxb.at[0])
            pltpu.sync_copy(p_hbm.at[c, pl.ds(row * CH, CH)], payb)
            pltpu.async_copy(payb, shared.at[idxb.at[0]], sem,
                             add=True).wait()
            return carry

        lax.fori_loop(0, npt, step, 0)
        plsc.subcore_barrier()
        pltpu.sync_copy(shared.at[pl.ds(s * rpt, rpt)],
                        o_hbm.at[c, pl.ds(s * rpt, rpt)])

    return pl.kernel(
        body,
        out_type=jax.ShapeDtypeStruct((NC, nd_t, w), jnp.float32),
        mesh=_mesh(),
        scratch_types=[
            pltpu.VMEM((1, CH), jnp.int32),
            pltpu.VMEM((CH, w), jnp.float32),
            pltpu.VMEM((CH, w), jnp.float32),
            pltpu.VMEM_SHARED((nd_t, w), jnp.float32),
            pltpu.SemaphoreType.DMA,
        ],
    )(pay, dst_flat, zchunk)


def _tc_proj(x, wcat, bcat, hc):
    ns, ci = x.shape
    g = 1000
    ng = ns // g
    wout = max(hc, 128)

    def body(x_ref, w_ref, b_ref, q_ref, k_ref, v_ref, s_ref):
        y = jnp.dot(x_ref[...], w_ref[...],
                    preferred_element_type=jnp.float32) + b_ref[...]
        for hh, oref in enumerate((q_ref, k_ref, v_ref, s_ref)):
            part = y[:, hh * hc:(hh + 1) * hc]
            if wout > hc:
                part = jnp.concatenate(
                    [part, jnp.zeros((g, wout - hc), jnp.float32)], axis=1)
            oref[...] = part

    oshape = jax.ShapeDtypeStruct((ns, wout), jnp.float32)
    return pl.pallas_call(
        body,
        grid=(ng,),
        in_specs=[
            pl.BlockSpec((g, ci), lambda i: (i, 0)),
            pl.BlockSpec((ci, 4 * hc), lambda i: (0, 0)),
            pl.BlockSpec((1, 4 * hc), lambda i: (0, 0)),
        ],
        out_specs=[pl.BlockSpec((g, wout), lambda i: (i, 0))] * 4,
        out_shape=[oshape] * 4,
    )(x, wcat, bcat)


def _tc_edge(qd, ks, vs, hdot, cdot, hpay, cpay, w):
    ep, hc = qd.shape
    g = 1024
    ng = ep // g
    scale = 1.0 / math.sqrt(cdot)

    def body(q_ref, k_ref, v_ref, o_ref):
        t = q_ref[...] * k_ref[...]
        v = v_ref[...]
        exs = []
        for hh in range(hdot):
            lg = jnp.sum(t[:, hh * cdot:(hh + 1) * cdot], axis=1,
                         keepdims=True) * scale
            exs.append(jnp.exp(lg))
        outs = []
        for p in range(hpay):
            ex = exs[p] if hdot > 1 else exs[0]
            pay = jnp.concatenate(
                [v[:, p * cpay:(p + 1) * cpay] * ex, ex,
                 jnp.zeros((g, w - cpay - 1), jnp.float32)], axis=1)
            outs.append(pay[None])
        o_ref[...] = (jnp.concatenate(outs, axis=0) if hpay > 1
                      else outs[0])

    return pl.pallas_call(
        body,
        grid=(ng,),
        in_specs=[pl.BlockSpec((g, hc), lambda i: (i, 0))] * 3,
        out_specs=pl.BlockSpec((hpay, g, w), lambda i: (0, i, 0)),
        out_shape=jax.ShapeDtypeStruct((hpay, ep, w), jnp.float32),
    )(qd, ks, vs)


def _tc_combine(tbl, xr, wbo, wbx, h, c, nd, want_bn):
    w = tbl.shape[2]
    g = 1000 if nd % 1000 == 0 else nd
    ng = nd // g
    hc = h * c

    def body(t_ref, x_ref, wo_ref, wx_ref, *refs):
        t = t_ref[...]
        if h == 2:
            parts = [t[hh][:, :c] / (t[hh][:, c:c + 1] + 1e-16)
                     for hh in range(2)]
            out = jnp.concatenate(parts, axis=1)
        else:
            tt = t[0] + t[1]
            out = tt[:, :c] / (tt[:, c:c + 1] + 1e-16)
        xr_ = x_ref[...]
        beta = jax.nn.sigmoid(
            jnp.dot(out, wo_ref[...], preferred_element_type=jnp.float32)
            + jnp.dot(xr_, wx_ref[...], preferred_element_type=jnp.float32))
        y = beta * xr_ + (1.0 - beta) * out
        refs[0][...] = y
        if want_bn:
            st_ref, acc = refs[1], refs[2]
            i = pl.program_id(0)

            @pl.when(i == 0)
            def _():
                acc[...] = jnp.zeros_like(acc)

            acc[0:1, :] += jnp.sum(y, axis=0, keepdims=True)
            acc[1:2, :] += jnp.sum(y * y, axis=0, keepdims=True)

            @pl.when(i == ng - 1)
            def _():
                st_ref[...] = acc[...]

    in_specs = [
        pl.BlockSpec((2, g, w), lambda i: (0, i, 0)),
        pl.BlockSpec((g, hc), lambda i: (i, 0)),
        pl.BlockSpec((hc, 1), lambda i: (0, 0)),
        pl.BlockSpec((hc, 1), lambda i: (0, 0)),
    ]
    if want_bn:
        return pl.pallas_call(
            body,
            grid=(ng,),
            in_specs=in_specs,
            out_specs=[pl.BlockSpec((g, hc), lambda i: (i, 0)),
                       pl.BlockSpec((8, 128), lambda i: (0, 0))],
            out_shape=[jax.ShapeDtypeStruct((nd, hc), jnp.float32),
                       jax.ShapeDtypeStruct((8, 128), jnp.float32)],
            scratch_shapes=[pltpu.VMEM((8, 128), jnp.float32)],
        )(tbl, xr, wbo, wbx)
    return pl.pallas_call(
        body,
        grid=(ng,),
        in_specs=in_specs,
        out_specs=pl.BlockSpec((g, hc), lambda i: (i, 0)),
        out_shape=jax.ShapeDtypeStruct((nd, hc), jnp.float32),
    )(tbl, xr, wbo, wbx)


def _tc_bn_elu(y, stats, gamma, beta, nd):
    g = 1000
    ng = nd // g
    inv_n = 1.0 / nd

    def body(y_ref, st_ref, g_ref, b_ref, o_ref):
        m = st_ref[0:1, :] * inv_n
        var = st_ref[1:2, :] * inv_n - m * m
        xh = (y_ref[...] - m) * lax.rsqrt(var + 1e-5) * g_ref[...] + b_ref[...]
        o_ref[...] = jnp.where(xh > 0, xh, jnp.exp(jnp.minimum(xh, 0.0)) - 1.0)

    return pl.pallas_call(
        body,
        grid=(ng,),
        in_specs=[
            pl.BlockSpec((g, 128), lambda i: (i, 0)),
            pl.BlockSpec((8, 128), lambda i: (0, 0)),
            pl.BlockSpec((1, 128), lambda i: (0, 0)),
            pl.BlockSpec((1, 128), lambda i: (0, 0)),
        ],
        out_specs=pl.BlockSpec((g, 128), lambda i: (i, 0)),
        out_shape=jax.ShapeDtypeStruct((nd, 128), jnp.float32),
    )(y, stats, gamma, beta)


def _tc_final(e0, e1, rl, w, b, u):
    def body(e0_ref, e1_ref, rl_ref, w_ref, b_ref, u_ref, o_ref):
        s0 = e0_ref[...] * rl_ref[0, 0]
        s1 = e1_ref[...] * rl_ref[1, 0]
        vm0 = jnp.tanh(jnp.dot(s0, w_ref[...],
                               preferred_element_type=jnp.float32) + b_ref[...])
        vm1 = jnp.tanh(jnp.dot(s1, w_ref[...],
                               preferred_element_type=jnp.float32) + b_ref[...])
        vu0 = jnp.dot(vm0, u_ref[...], preferred_element_type=jnp.float32)
        vu1 = jnp.dot(vm1, u_ref[...], preferred_element_type=jnp.float32)
        m = jnp.maximum(vu0, vu1)
        a0 = jnp.exp(vu0 - m)
        a1 = jnp.exp(vu1 - m)
        o_ref[...] = (a0 * s0 + a1 * s1) / (a0 + a1)

    return pl.pallas_call(
        body,
        grid=(1,),
        in_specs=[
            pl.BlockSpec((NB, 128), lambda i: (0, 0)),
            pl.BlockSpec((NB, 128), lambda i: (0, 0)),
            pl.BlockSpec(memory_space=pltpu.SMEM),
            pl.BlockSpec((128, 128), lambda i: (0, 0)),
            pl.BlockSpec((1, 128), lambda i: (0, 0)),
            pl.BlockSpec((128, 1), lambda i: (0, 0)),
        ],
        out_specs=pl.BlockSpec((NB, 128), lambda i: (0, 0)),
        out_shape=jax.ShapeDtypeStruct((NB, 128), jnp.float32),
    )(e0, e1, rl, w, b, u)


def _layer(x, src, dst, p, l):
    ci, co, h = LAYER_SHAPES[l]
    hc = h * co
    nd = TGT[l]
    epad = EPAD[l]
    e0 = E_SIZES[l]
    hpay = 2 if (h == 2 or co == 128) else 1
    cpay = 64
    wpay = 128

    wcat = jnp.concatenate([p["Wq"], p["Wk"], p["Wv"], p["Ws"]], axis=1)
    bcat = jnp.concatenate([p["bq"], p["bk"], p["bv"], p["bs"]])[None]
    q, k, v, s = _tc_proj(x, wcat, bcat, hc)

    npad = epad - e0
    src_p = jnp.pad(src, (0, npad))
    dst_g = jnp.pad(dst, (0, npad))
    dst_s = jnp.pad(dst, (0, npad), constant_values=N1 + N2)

    qd, ks, vs = _sc_gather3(q, k, v, dst_g, src_p)
    pay = _tc_edge(qd, ks, vs, h, co, hpay, cpay, wpay)

    zchunk = jnp.zeros((CH, wpay), jnp.float32)
    if hpay == 2:
        bounds = ((0, L0_SPLIT), (L0_SPLIT, nd)) if nd > L0_SPLIT \
            else ((0, nd),)
        halves = []
        for lo, hi in bounds:
            size = hi - lo
            spread = NDT[l] - size
            dmp = size + (dst_s % spread)
            dst_p = jnp.where((dst_s >= lo) & (dst_s < hi), dst_s - lo, dmp)
            dst_cat = jnp.concatenate([dst_p, dst_p])
            tbl_p = _sc_scatter(pay, dst_cat, zchunk, NDT[l], wpay)
            halves.append(tbl_p[:, :size])
        tbl = halves[0] if len(halves) == 1 else \
            jnp.concatenate(halves, axis=1)
    else:
        spread = NDT[l] - nd
        pad_rows = nd + (jnp.arange(npad, dtype=jnp.int32) % spread)
        dst_cat = jnp.concatenate([dst, pad_rows])
        pay_sc = pay.reshape(2, epad // 2, wpay)
        tbl = _sc_scatter(pay_sc, dst_cat, zchunk, NDT[l], wpay)

    wb = p["Wb"]
    wbo = wb[:hc] + wb[2 * hc:]
    wbx = wb[hc:2 * hc] - wb[2 * hc:]
    xr = s[:nd, :hc]
    return _tc_combine(tbl, xr, wbo, wbx, hpay, cpay, nd, want_bn=(l == 0))


def kernel(features, RL_thresholds, params, n_ids_0, n_ids_1,
           src_r0_l0, dst_r0_l0, src_r0_l1, dst_r0_l1, src_r0_l2, dst_r0_l2,
           src_r1_l0, dst_r1_l0, src_r1_l1, dst_r1_l1, src_r1_l2, dst_r1_l2):
    edges = [
        [(src_r0_l0, dst_r0_l0), (src_r0_l1, dst_r0_l1), (src_r0_l2, dst_r0_l2)],
        [(src_r1_l0, dst_r1_l0), (src_r1_l1, dst_r1_l1), (src_r1_l2, dst_r1_l2)],
    ]
    idx_cat = jnp.concatenate([
        jnp.pad(n_ids_0, (0, IDPAD - N0)),
        jnp.pad(n_ids_1, (0, IDPAD - N0)),
    ])
    xall = _sc_gather(features, idx_cat)
    xs = [xall[:N0], xall[IDPAD:IDPAD + N0]]

    embeds = []
    for r in range(2):
        x = xs[r]
        for l in range(3):
            srcl, dstl = edges[r][l]
            p = params["r%d_l%d" % (r, l)]
            res = _layer(x, srcl, dstl, p, l)
            if l == 0:
                y, stats = res
                bnp = params["bn_r%d" % r]
                x = _tc_bn_elu(y, stats, bnp["gamma"][None],
                               bnp["beta"][None], TGT[l])
            else:
                x = res
        embeds.append(x)

    ap = params["attn"]
    return _tc_final(embeds[0], embeds[1], RL_thresholds,
                     ap["w"], ap["b"][None], ap["u"][:, None])


# speedup vs baseline: 20.0709x; 1.1316x over previous
import functools
import math

import jax
import jax.numpy as jnp
from jax import lax
from jax.experimental import pallas as pl
from jax.experimental.pallas import tpu as pltpu
from jax.experimental.pallas import tpu_sc as plsc

NC, NS = 2, 16
NW = NC * NS
CH = 128

N_NODES = 100000
D = 128
N0, N1, N2, NB = 50000, 20000, 5000, 1024
E_SIZES = [320000, 80000, 16384]
EPAD = [323584, 81920, 16384]
TGT = [N1, N2, NB]
NDT = [12032, 6144, 2048]
L0_SPLIT = 10240
LAYER_SHAPES = [(128, 64, 2), (128, 64, 1), (64, 128, 1)]
IDPAD = 51200


def _mesh():
    return plsc.VectorSubcoreMesh(core_axis_name="c", subcore_axis_name="s")


def _sc_gather(table, idx):
    nch = idx.shape[0] // CH
    Dd = table.shape[1]
    npt = nch // NW

    def body(t_hbm, i_hbm, o_hbm, *scr):
        c = lax.axis_index("c")
        s = lax.axis_index("s")
        wid = s * NC + c
        bufs = (scr[:3], scr[3:6])

        def fire(b, j):
            idxb, rows, sem = bufs[b]
            row = wid * npt + j
            pltpu.sync_copy(i_hbm.at[pl.ds(row * CH, CH)], idxb)
            pltpu.async_copy(t_hbm.at[idxb], rows, sem)

        def drain_write(b, j):
            idxb, rows, sem = bufs[b]
            row = wid * npt + j
            pltpu.make_async_copy(t_hbm.at[pl.ds(0, CH)], rows, sem).wait()
            pltpu.sync_copy(rows, o_hbm.at[pl.ds(row * CH, CH)])

        fire(0, 0)
        if npt > 1:
            fire(1, 1)

        def pair(j2, carry):
            for b in (0, 1):
                j = 2 * j2 + b

                @pl.when(j < npt)
                def _():
                    drain_write(b, j)

                @pl.when(j + 2 < npt)
                def _():
                    fire(b, j + 2)
            return carry

        lax.fori_loop(0, (npt + 1) // 2, pair, 0)

    buf = [
        pltpu.VMEM((CH,), jnp.int32),
        pltpu.VMEM((CH, Dd), jnp.float32),
        pltpu.SemaphoreType.DMA,
    ]
    return pl.kernel(
        body,
        out_type=jax.ShapeDtypeStruct((nch * CH, Dd), jnp.float32),
        mesh=_mesh(),
        scratch_types=buf + buf,
    )(table, idx)


def _sc_gather3(qt, kt, vt, dst1, src1):
    nch = dst1.shape[0] // CH
    hc = qt.shape[1]
    npt = nch // NW
    oshape = jax.ShapeDtypeStruct((nch * CH, hc), jnp.float32)

    def body(q_hbm, k_hbm, v_hbm, d_hbm, s_hbm, qd_hbm, ks_hbm, vs_hbm,
             *scr):
        c = lax.axis_index("c")
        s = lax.axis_index("s")
        wid = s * NC + c
        bufs = (scr[:6], scr[6:12])

        def fire(b, j):
            di, si, qb, kb, vb, sem = bufs[b]
            row = wid * npt + j
            pltpu.sync_copy(d_hbm.at[pl.ds(row * CH, CH)], di)
            pltpu.sync_copy(s_hbm.at[pl.ds(row * CH, CH)], si)
            pltpu.async_copy(q_hbm.at[di], qb, sem)
            pltpu.async_copy(k_hbm.at[si], kb, sem)
            pltpu.async_copy(v_hbm.at[si], vb, sem)

        def drain_write(b, j):
            di, si, qb, kb, vb, sem = bufs[b]
            row = wid * npt + j
            pltpu.make_async_copy(q_hbm.at[pl.ds(0, CH)], qb, sem).wait()
            pltpu.make_async_copy(k_hbm.at[pl.ds(0, CH)], kb, sem).wait()
            pltpu.make_async_copy(v_hbm.at[pl.ds(0, CH)], vb, sem).wait()
            e0 = row * CH
            pltpu.sync_copy(qb, qd_hbm.at[pl.ds(e0, CH)])
            pltpu.sync_copy(kb, ks_hbm.at[pl.ds(e0, CH)])
            pltpu.sync_copy(vb, vs_hbm.at[pl.ds(e0, CH)])

        fire(0, 0)
        if npt > 1:
            fire(1, 1)

        def pair(j2, carry):
            for b in (0, 1):
                j = 2 * j2 + b

                @pl.when(j < npt)
                def _():
                    drain_write(b, j)

                @pl.when(j + 2 < npt)
                def _():
                    fire(b, j + 2)
            return carry

        lax.fori_loop(0, (npt + 1) // 2, pair, 0)

    buf = [
        pltpu.VMEM((CH,), jnp.int32),
        pltpu.VMEM((CH,), jnp.int32),
        pltpu.VMEM((CH, hc), jnp.float32),
        pltpu.VMEM((CH, hc), jnp.float32),
        pltpu.VMEM((CH, hc), jnp.float32),
        pltpu.SemaphoreType.DMA,
    ]
    return pl.kernel(
        body,
        out_type=[oshape, oshape, oshape],
        mesh=_mesh(),
        scratch_types=buf + buf,
    )(qt, kt, vt, dst1, src1)


def _sc_scatter(pay, dst_flat, zchunk, nd_t, w):
    eh = pay.shape[1]
    npt = eh // (NS * CH)
    rpt = nd_t // NS
    nz = rpt // CH
    rem = rpt % CH

    def body(p_hbm, d_hbm, z_hbm, o_hbm, idxb, payb, zb, shared, sem):
        c = lax.axis_index("c")
        s = lax.axis_index("s")
        pltpu.sync_copy(z_hbm, zb)

        def zstep(i, carry):
            pltpu.sync_copy(zb, shared.at[pl.ds(s * rpt + i * CH, CH)])
            return carry

        lax.fori_loop(0, nz, zstep, 0)
        if rem:
            pltpu.sync_copy(zb.at[pl.ds(0, rem)],
                            shared.at[pl.ds(s * rpt + nz * CH, rem)])
        plsc.subcore_barrier()

        def step(j, carry):
            row = s * npt + j
            pltpu.sync_copy(d_hbm.at[pl.ds(c * eh + row * CH, CH)],
                            idxb.at[0])
            pltpu.sync_copy(p_hbm.at[c, pl.ds(row * CH, CH)], payb)
            pltpu.async_copy(payb, shared.at[idxb.at[0]], sem,
                             add=True).wait()
            return carry

        lax.fori_loop(0, npt, step, 0)
        plsc.subcore_barrier()
        pltpu.sync_copy(shared.at[pl.ds(s * rpt, rpt)],
                        o_hbm.at[c, pl.ds(s * rpt, rpt)])

    return pl.kernel(
        body,
        out_type=jax.ShapeDtypeStruct((NC, nd_t, w), jnp.float32),
        mesh=_mesh(),
        scratch_types=[
            pltpu.VMEM((1, CH), jnp.int32),
            pltpu.VMEM((CH, w), jnp.float32),
            pltpu.VMEM((CH, w), jnp.float32),
            pltpu.VMEM_SHARED((nd_t, w), jnp.float32),
            pltpu.SemaphoreType.DMA,
        ],
    )(pay, dst_flat, zchunk)


def _tc_proj(x, wcat, bcat, hc):
    ns, ci = x.shape
    g = 1000
    ng = ns // g
    wout = max(hc, 128)

    def body(x_ref, w_ref, b_ref, q_ref, k_ref, v_ref, s_ref):
        y = jnp.dot(x_ref[...], w_ref[...],
                    preferred_element_type=jnp.float32) + b_ref[...]
        for hh, oref in enumerate((q_ref, k_ref, v_ref, s_ref)):
            part = y[:, hh * hc:(hh + 1) * hc]
            if wout > hc:
                part = jnp.concatenate(
                    [part, jnp.zeros((g, wout - hc), jnp.float32)], axis=1)
            oref[...] = part

    oshape = jax.ShapeDtypeStruct((ns, wout), jnp.float32)
    return pl.pallas_call(
        body,
        grid=(ng,),
        in_specs=[
            pl.BlockSpec((g, ci), lambda i: (i, 0)),
            pl.BlockSpec((ci, 4 * hc), lambda i: (0, 0)),
            pl.BlockSpec((1, 4 * hc), lambda i: (0, 0)),
        ],
        out_specs=[pl.BlockSpec((g, wout), lambda i: (i, 0))] * 4,
        out_shape=[oshape] * 4,
    )(x, wcat, bcat)


def _tc_edge(qd, ks, vs, hdot, cdot, hpay, cpay, w):
    ep, hc = qd.shape
    g = 1024
    ng = ep // g
    scale = 1.0 / math.sqrt(cdot)

    def body(q_ref, k_ref, v_ref, o_ref):
        t = q_ref[...] * k_ref[...]
        v = v_ref[...]
        exs = []
        for hh in range(hdot):
            lg = jnp.sum(t[:, hh * cdot:(hh + 1) * cdot], axis=1,
                         keepdims=True) * scale
            exs.append(jnp.exp(lg))
        outs = []
        for p in range(hpay):
            ex = exs[p] if hdot > 1 else exs[0]
            pay = jnp.concatenate(
                [v[:, p * cpay:(p + 1) * cpay] * ex, ex,
                 jnp.zeros((g, w - cpay - 1), jnp.float32)], axis=1)
            outs.append(pay[None])
        o_ref[...] = (jnp.concatenate(outs, axis=0) if hpay > 1
                      else outs[0])

    return pl.pallas_call(
        body,
        grid=(ng,),
        in_specs=[pl.BlockSpec((g, hc), lambda i: (i, 0))] * 3,
        out_specs=pl.BlockSpec((hpay, g, w), lambda i: (0, i, 0)),
        out_shape=jax.ShapeDtypeStruct((hpay, ep, w), jnp.float32),
    )(qd, ks, vs)


def _tc_combine(tbl, xr, wbo, wbx, h, c, nd, want_bn):
    w = tbl.shape[2]
    g = 1000 if nd % 1000 == 0 else nd
    ng = nd // g
    hc = h * c

    def body(t_ref, x_ref, wo_ref, wx_ref, *refs):
        t = t_ref[...]
        if h == 2:
            parts = [t[hh][:, :c] / (t[hh][:, c:c + 1] + 1e-16)
                     for hh in range(2)]
            out = jnp.concatenate(parts, axis=1)
        else:
            tt = t[0] + t[1]
            out = tt[:, :c] / (tt[:, c:c + 1] + 1e-16)
        xr_ = x_ref[...]
        beta = jax.nn.sigmoid(
            jnp.dot(out, wo_ref[...], preferred_element_type=jnp.float32)
            + jnp.dot(xr_, wx_ref[...], preferred_element_type=jnp.float32))
        y = beta * xr_ + (1.0 - beta) * out
        refs[0][...] = y
        if want_bn:
            st_ref, acc = refs[1], refs[2]
            i = pl.program_id(0)

            @pl.when(i == 0)
            def _():
                acc[...] = jnp.zeros_like(acc)

            acc[0:1, :] += jnp.sum(y, axis=0, keepdims=True)
            acc[1:2, :] += jnp.sum(y * y, axis=0, keepdims=True)

            @pl.when(i == ng - 1)
            def _():
                st_ref[...] = acc[...]

    in_specs = [
        pl.BlockSpec((2, g, w), lambda i: (0, i, 0)),
        pl.BlockSpec((g, hc), lambda i: (i, 0)),
        pl.BlockSpec((hc, 1), lambda i: (0, 0)),
        pl.BlockSpec((hc, 1), lambda i: (0, 0)),
    ]
    if want_bn:
        return pl.pallas_call(
            body,
            grid=(ng,),
            in_specs=in_specs,
            out_specs=[pl.BlockSpec((g, hc), lambda i: (i, 0)),
                       pl.BlockSpec((8, 128), lambda i: (0, 0))],
            out_shape=[jax.ShapeDtypeStruct((nd, hc), jnp.float32),
                       jax.ShapeDtypeStruct((8, 128), jnp.float32)],
            scratch_shapes=[pltpu.VMEM((8, 128), jnp.float32)],
        )(tbl, xr, wbo, wbx)
    return pl.pallas_call(
        body,
        grid=(ng,),
        in_specs=in_specs,
        out_specs=pl.BlockSpec((g, hc), lambda i: (i, 0)),
        out_shape=jax.ShapeDtypeStruct((nd, hc), jnp.float32),
    )(tbl, xr, wbo, wbx)


def _tc_bn_elu(y, stats, gamma, beta, nd):
    g = 1000
    ng = nd // g
    inv_n = 1.0 / nd

    def body(y_ref, st_ref, g_ref, b_ref, o_ref):
        m = st_ref[0:1, :] * inv_n
        var = st_ref[1:2, :] * inv_n - m * m
        xh = (y_ref[...] - m) * lax.rsqrt(var + 1e-5) * g_ref[...] + b_ref[...]
        o_ref[...] = jnp.where(xh > 0, xh, jnp.exp(jnp.minimum(xh, 0.0)) - 1.0)

    return pl.pallas_call(
        body,
        grid=(ng,),
        in_specs=[
            pl.BlockSpec((g, 128), lambda i: (i, 0)),
            pl.BlockSpec((8, 128), lambda i: (0, 0)),
            pl.BlockSpec((1, 128), lambda i: (0, 0)),
            pl.BlockSpec((1, 128), lambda i: (0, 0)),
        ],
        out_specs=pl.BlockSpec((g, 128), lambda i: (i, 0)),
        out_shape=jax.ShapeDtypeStruct((nd, 128), jnp.float32),
    )(y, stats, gamma, beta)


def _tc_final(e0, e1, rl, w, b, u):
    def body(e0_ref, e1_ref, rl_ref, w_ref, b_ref, u_ref, o_ref):
        s0 = e0_ref[...] * rl_ref[0, 0]
        s1 = e1_ref[...] * rl_ref[1, 0]
        vm0 = jnp.tanh(jnp.dot(s0, w_ref[...],
                               preferred_element_type=jnp.float32) + b_ref[...])
        vm1 = jnp.tanh(jnp.dot(s1, w_ref[...],
                               preferred_element_type=jnp.float32) + b_ref[...])
        vu0 = jnp.dot(vm0, u_ref[...], preferred_element_type=jnp.float32)
        vu1 = jnp.dot(vm1, u_ref[...], preferred_element_type=jnp.float32)
        m = jnp.maximum(vu0, vu1)
        a0 = jnp.exp(vu0 - m)
        a1 = jnp.exp(vu1 - m)
        o_ref[...] = (a0 * s0 + a1 * s1) / (a0 + a1)

    return pl.pallas_call(
        body,
        grid=(1,),
        in_specs=[
            pl.BlockSpec((NB, 128), lambda i: (0, 0)),
            pl.BlockSpec((NB, 128), lambda i: (0, 0)),
            pl.BlockSpec(memory_space=pltpu.SMEM),
            pl.BlockSpec((128, 128), lambda i: (0, 0)),
            pl.BlockSpec((1, 128), lambda i: (0, 0)),
            pl.BlockSpec((128, 1), lambda i: (0, 0)),
        ],
        out_specs=pl.BlockSpec((NB, 128), lambda i: (0, 0)),
        out_shape=jax.ShapeDtypeStruct((NB, 128), jnp.float32),
    )(e0, e1, rl, w, b, u)


def _layer(x, src, dst, p, l):
    ci, co, h = LAYER_SHAPES[l]
    hc = h * co
    nd = TGT[l]
    epad = EPAD[l]
    e0 = E_SIZES[l]
    hpay = 2 if (h == 2 or co == 128) else 1
    cpay = 64
    wpay = 128

    wcat = jnp.concatenate([p["Wq"], p["Wk"], p["Wv"], p["Ws"]], axis=1)
    bcat = jnp.concatenate([p["bq"], p["bk"], p["bv"], p["bs"]])[None]
    q, k, v, s = _tc_proj(x, wcat, bcat, hc)

    npad = epad - e0
    src_p = jnp.pad(src, (0, npad))
    dst_g = jnp.pad(dst, (0, npad))
    dst_s = jnp.pad(dst, (0, npad), constant_values=N1 + N2)

    qd, ks, vs = _sc_gather3(q, k, v, dst_g, src_p)
    pay = _tc_edge(qd, ks, vs, h, co, hpay, cpay, wpay)

    zchunk = jnp.zeros((CH, wpay), jnp.float32)
    if hpay == 2:
        bounds = ((0, L0_SPLIT), (L0_SPLIT, nd)) if nd > L0_SPLIT \
            else ((0, nd),)
        halves = []
        for lo, hi in bounds:
            size = hi - lo
            spread = NDT[l] - size
            dmp = size + (dst_s % spread)
            dst_p = jnp.where((dst_s >= lo) & (dst_s < hi), dst_s - lo, dmp)
            dst_cat = jnp.concatenate([dst_p, dst_p])
            tbl_p = _sc_scatter(pay, dst_cat, zchunk, NDT[l], wpay)
            halves.append(tbl_p[:, :size])
        tbl = halves[0] if len(halves) == 1 else \
            jnp.concatenate(halves, axis=1)
    else:
        spread = NDT[l] - nd
        pad_rows = nd + (jnp.arange(npad, dtype=jnp.int32) % spread)
        dst_cat = jnp.concatenate([dst, pad_rows])
        pay_sc = pay.reshape(2, epad // 2, wpay)
        tbl = _sc_scatter(pay_sc, dst_cat, zchunk, NDT[l], wpay)

    wb = p["Wb"]
    wbo = wb[:hc] + wb[2 * hc:]
    wbx = wb[hc:2 * hc] - wb[2 * hc:]
    xr = s[:nd, :hc]
    return _tc_combine(tbl, xr, wbo, wbx, hpay, cpay, nd, want_bn=(l == 0))


def kernel(features, RL_thresholds, params, n_ids_0, n_ids_1,
           src_r0_l0, dst_r0_l0, src_r0_l1, dst_r0_l1, src_r0_l2, dst_r0_l2,
           src_r1_l0, dst_r1_l0, src_r1_l1, dst_r1_l1, src_r1_l2, dst_r1_l2):
    edges = [
        [(src_r0_l0, dst_r0_l0), (src_r0_l1, dst_r0_l1), (src_r0_l2, dst_r0_l2)],
        [(src_r1_l0, dst_r1_l0), (src_r1_l1, dst_r1_l1), (src_r1_l2, dst_r1_l2)],
    ]
    idx_cat = jnp.concatenate([
        jnp.pad(n_ids_0, (0, IDPAD - N0)),
        jnp.pad(n_ids_1, (0, IDPAD - N0)),
    ])
    xall = _sc_gather(features, idx_cat)
    xs = [xall[:N0], xall[IDPAD:IDPAD + N0]]

    embeds = []
    for r in range(2):
        x = xs[r]
        for l in range(3):
            srcl, dstl = edges[r][l]
            p = params["r%d_l%d" % (r, l)]
            res = _layer(x, srcl, dstl, p, l)
            if l == 0:
                y, stats = res
                bnp = params["bn_r%d" % r]
                x = _tc_bn_elu(y, stats, bnp["gamma"][None],
                               bnp["beta"][None], TGT[l])
            else:
                x = res
        embeds.append(x)

    ap = params["attn"]
    return _tc_final(embeds[0], embeds[1], RL_thresholds,
                     ap["w"], ap["b"][None], ap["u"][:, None])
